# Initial kernel scaffold; baseline (speedup 1.0000x reference)
import jax
import jax.numpy as jnp
from jax.experimental import pallas as pl


def kernel(idx, table):
    raise NotImplementedError("write your pallas kernel here")


# speedup vs baseline: 19.2497x; 19.2497x over previous
import functools

import jax
import jax.numpy as jnp
from jax import lax
from jax.experimental import pallas as pl
from jax.experimental.pallas import tpu as pltpu
from jax.experimental.pallas import tpu_sc as plsc

D = 128
NC, NS = 2, 16
NW = NC * NS
CHUNK = 128
NBUF = 2


def _softmax_rows_body(tab_ref, out_ref):
    x = tab_ref[...]
    m = jnp.max(x, axis=-1, keepdims=True)
    e = jnp.exp(x - m)
    out_ref[...] = e / jnp.sum(e, axis=-1, keepdims=True)


def _softmax_rows(table):
    v, d = table.shape
    blk = 5000
    assert v % blk == 0
    return pl.pallas_call(
        _softmax_rows_body,
        grid=(v // blk,),
        in_specs=[pl.BlockSpec((blk, d), lambda i: (i, 0))],
        out_specs=pl.BlockSpec((blk, d), lambda i: (i, 0)),
        out_shape=jax.ShapeDtypeStruct((v, d), jnp.float32),
    )(table)


def _sc_gather(sm_table, idx3):
    n_chunks = idx3.shape[1]
    per_w = n_chunks * CHUNK
    b_total = NW * per_w
    mesh = plsc.VectorSubcoreMesh(
        core_axis_name="c", subcore_axis_name="s", num_cores=NC, num_subcores=NS
    )

    @functools.partial(
        pl.kernel,
        mesh=mesh,
        out_type=jax.ShapeDtypeStruct((b_total, D), jnp.float32),
        scratch_types=[
            pltpu.VMEM((n_chunks, CHUNK), jnp.int32),
            pltpu.VMEM((NBUF, CHUNK, D), jnp.float32),
            pltpu.SemaphoreType.DMA((NBUF,)),
            pltpu.SemaphoreType.DMA((NBUF,)),
        ],
    )
    def gather_kernel(table_hbm, idx_hbm, out_hbm, idx_v, rows_v, sem_in, sem_out):
        wid = lax.axis_index("s") * NC + lax.axis_index("c")
        base = wid * per_w

        pltpu.sync_copy(idx_hbm.at[wid], idx_v)

        def start_in(j, b):
            pltpu.make_async_copy(
                table_hbm.at[idx_v.at[j]], rows_v.at[b], sem_in.at[b]
            ).start()

        def wait_in(b):
            pltpu.make_async_copy(
                table_hbm.at[idx_v.at[0]], rows_v.at[b], sem_in.at[b]
            ).wait()

        def start_out(j, b):
            pltpu.make_async_copy(
                rows_v.at[b], out_hbm.at[pl.ds(base + j * CHUNK, CHUNK)], sem_out.at[b]
            ).start()

        def wait_out(j, b):
            pltpu.make_async_copy(
                rows_v.at[b], out_hbm.at[pl.ds(base + j * CHUNK, CHUNK)], sem_out.at[b]
            ).wait()

        for b in range(NBUF):
            start_in(b, b)

        @pl.loop(0, n_chunks, step=NBUF)
        def _(g):
            for b in range(NBUF):
                j = g + b
                wait_in(b)
                start_out(j, b)
                wait_out(j, b)

                @pl.when(j + NBUF < n_chunks)
                def _():
                    start_in(j + NBUF, b)

    return gather_kernel(sm_table, idx3)


@jax.jit
def kernel(idx, table):
    batch, hist = idx.shape
    b_total = batch * hist
    per_w = b_total // NW
    n_chunks = per_w // CHUNK
    sm_table = _softmax_rows(table)
    idx3 = idx.reshape(NW, n_chunks, CHUNK).astype(jnp.int32)
    out = _sc_gather(sm_table, idx3)
    return out.reshape(batch, hist, D)
